# baseline (speedup 1.0000x reference)
import functools
import math

import jax
import jax.numpy as jnp
from jax import lax
from jax.experimental.layout import Format, Layout
from jax.experimental import pallas as pl
from jax.experimental.pallas import tpu as pltpu
from jax.experimental.pallas import tpu_sc as plsc

VOCAB_SIZE = 100000
D_MODEL = 128
BATCH = 4096
SEQ_LEN = 50
SCALE = math.sqrt(D_MODEL)

NC = 2
NS = 16
NW = NC * NS

B_PER_W = BATCH // NW
CHUNK_B = 2
CHUNK = CHUNK_B * SEQ_LEN
N_CHUNKS = B_PER_W // CHUNK_B
NBUF = 4


def _body(x_hbm, w_hbm, out_hbm, idx_v, rows_v, gsem, ssem):
    wid = lax.axis_index("s") * NC + lax.axis_index("c")
    b_base = wid * B_PER_W
    pltpu.sync_copy(x_hbm.at[wid], idx_v)

    def gather(c, buf):
        return pltpu.async_copy(w_hbm.at[idx_v.at[c]], rows_v.at[buf], gsem)

    def wait_gather(c, buf):
        pltpu.make_async_copy(
            w_hbm.at[idx_v.at[c]], rows_v.at[buf], gsem
        ).wait()

    def scatter(c, buf, wait):
        for i in range(CHUNK_B):
            src = rows_v.at[buf, pl.ds(i * SEQ_LEN, SEQ_LEN)]
            dst = out_hbm.at[b_base + c * CHUNK_B + i]
            if wait:
                pltpu.make_async_copy(src, dst, ssem).wait()
            else:
                pltpu.async_copy(src, dst, ssem)

    for b in range(NBUF - 1):
        gather(b, b)

    @pl.loop(0, N_CHUNKS, step=NBUF)
    def outer(c0):
        for k in range(NBUF):
            c = c0 + k
            prev = (k - 1) % NBUF
            wait_gather(c, k)

            @pl.when(c > 0)
            def _():
                scatter(c - 1, prev, wait=True)

            @pl.when(c + NBUF - 1 < N_CHUNKS)
            def _():
                gather(c + NBUF - 1, prev)

            @plsc.parallel_loop(0, CHUNK, unroll=4)
            def scale_row(r):
                for j in range(D_MODEL // 16):
                    rows_v[k, r, pl.ds(j * 16, 16)] = (
                        rows_v[k, r, pl.ds(j * 16, 16)] * SCALE
                    )

            scatter(c, k, wait=False)

    scatter(N_CHUNKS - 1, (N_CHUNKS - 1) % NBUF, wait=True)


def _impl(x, weight):
    xf = x.reshape(NW, N_CHUNKS, CHUNK)
    mesh = plsc.VectorSubcoreMesh(
        core_axis_name="c", subcore_axis_name="s", num_cores=NC, num_subcores=NS
    )
    return pl.kernel(
        _body,
        out_type=jax.ShapeDtypeStruct((BATCH, SEQ_LEN, D_MODEL), jnp.float32),
        mesh=mesh,
        scratch_types=[
            pltpu.VMEM((N_CHUNKS, CHUNK), jnp.int32),
            pltpu.VMEM((NBUF, CHUNK, D_MODEL), jnp.float32),
            pltpu.SemaphoreType.DMA,
            pltpu.SemaphoreType.DMA,
        ],
    )(xf, weight)


@functools.cache
def _jitted():
    fmt = Format(
        Layout(major_to_minor=(0, 1, 2), tiling=()),
        jax.sharding.SingleDeviceSharding(jax.devices()[0]),
    )
    return jax.jit(_impl, out_shardings=fmt)


def kernel(x, weight):
    return _jitted()(x, weight)
